# baseline (speedup 1.0000x reference)
import functools

import jax
import jax.numpy as jnp
from jax import lax
from jax.experimental import pallas as pl
from jax.experimental.pallas import tpu as pltpu
from jax.experimental.pallas import tpu_sc as plsc

NW = 32
NPW = 320
GB = 16
IDXSPAN = 2048
LANES = 16
D = 128
NFG = D // LANES


def _oddeven_network(length):
    pairs = []

    def merge(lo, n, r):
        step = r * 2
        if step < n:
            merge(lo, n, step)
            merge(lo + r, n, step)
            for i in range(lo + r, lo + n - r, step):
                pairs.append((i, i + r))
        else:
            pairs.append((lo, lo + r))

    def sort(lo, n):
        if n > 1:
            m = n // 2
            sort(lo, m)
            sort(lo + m, m)
            merge(lo, n, 1)

    sort(0, length)
    return pairs


def _prune_for_median(net, B):
    needed = set(range((B + 1) // 2))
    keep = []
    for (i, j) in reversed(net):
        if i in needed or j in needed:
            keep.append((i, j))
            needed.add(i)
            needed.add(j)
    keep.reverse()
    return keep


_NET64 = _oddeven_network(64)
_NET48 = [(i, j) for (i, j) in _NET64 if j < 48]
_NETS = {32: _prune_for_median(_oddeven_network(32), 32),
         48: _prune_for_median(_NET48, 48),
         64: _prune_for_median(_NET64, 64)}


def _matmul(x, W):
    rows = x.shape[0]

    def body(x_ref, w_ref, o_ref):
        o_ref[...] = jnp.dot(x_ref[...], w_ref[...],
                             preferred_element_type=jnp.float32)

    return pl.pallas_call(
        body,
        grid=(rows // 128,),
        in_specs=[pl.BlockSpec((128, x.shape[1]), lambda i: (i, 0)),
                  pl.BlockSpec(W.shape, lambda i: (0, 0))],
        out_specs=pl.BlockSpec((128, W.shape[1]), lambda i: (i, 0)),
        out_shape=jax.ShapeDtypeStruct((rows, W.shape[1]), jnp.float32),
    )(x, W)


def _sc_median(h, rows_pad, offs_pad, deg_pad, bias):
    npad = offs_pad.shape[0]
    NG = NPW // GB
    mesh = plsc.VectorSubcoreMesh(core_axis_name="c", subcore_axis_name="s",
                                  num_cores=2, num_subcores=16)

    @functools.partial(
        pl.kernel, mesh=mesh,
        out_type=jax.ShapeDtypeStruct((npad, D), jnp.float32),
        scratch_types=[
            pltpu.VMEM((NPW + 16,), jnp.int32),
            pltpu.VMEM((NPW + 16,), jnp.int32),
            pltpu.VMEM((2 * IDXSPAN,), jnp.int32),
            pltpu.VMEM((192, D), jnp.float32),
            pltpu.VMEM((64,), jnp.int32),
            pltpu.VMEM((64, D), jnp.float32),
            pltpu.VMEM((32, LANES), jnp.float32),
            pltpu.VMEM((GB, D), jnp.float32),
            pltpu.VMEM((D,), jnp.float32),
            pltpu.SemaphoreType.DMA,
            pltpu.SemaphoreType.DMA,
            pltpu.SemaphoreType.DMA,
        ])
    def kern(h_hbm, rows_hbm, offs_hbm, deg_hbm, bias_hbm, out_hbm,
             offs_v, deg_v, gidx_v, vals_v, fidx_v, fvals_v, srt_v,
             outg_v, bias_v, sem_gi, sem_gv, sem_f):
        cid = lax.axis_index("c")
        sid = lax.axis_index("s")
        wid = sid * 2 + (1 - cid)
        nbase = pl.multiple_of(wid * NPW, 8)
        pltpu.sync_copy(offs_hbm.at[pl.ds(nbase, NPW)],
                        offs_v.at[pl.ds(0, NPW)])
        pltpu.sync_copy(deg_hbm.at[pl.ds(nbase, NPW)],
                        deg_v.at[pl.ds(0, NPW)])
        pltpu.sync_copy(bias_hbm, bias_v)
        zf = bias_v[pl.ds(0, LANES)] * 0.0
        inf16 = zf + jnp.float32(jnp.inf)
        zero_i = lax.bitcast_convert_type(zf, jnp.int32)
        one_i = zero_i + 1
        zero_u = lax.bitcast_convert_type(zf, jnp.uint32)
        ffff_u = ~zero_u

        def rd(ref, i):
            return ref[pl.ds(i, 16)][0]

        def grp_base(g):
            return pl.multiple_of((rd(offs_v, g * GB) >> 3) << 3, 8)

        def node_params(nidx):
            start = rd(offs_v, nidx)
            d = rd(deg_v, nidx)
            a = pl.multiple_of((start >> 3) << 3, 8)
            skew = start - a
            m = skew + d
            a_rel = pl.multiple_of(a - grp_base(nidx >> 4), 8)
            use_grp = (m <= 64) & ((a_rel + 64) <= IDXSPAN)
            return d, a, skew, m, a_rel, use_grp

        def issue_gather(nidx):
            d, a, skew, m, a_rel, use_grp = node_params(nidx)

            @pl.when(use_grp)
            def _():
                goff = ((nidx >> 4) & 1) * IDXSPAN
                boff = (nidx % 3) * 64
                pltpu.async_copy(
                    h_hbm.at[gidx_v.at[
                        pl.ds(pl.multiple_of(goff + a_rel, 8), 64)]],
                    vals_v.at[pl.ds(pl.multiple_of(boff, 8), 64)],
                    sem_gv)

        pltpu.sync_copy(rows_hbm.at[pl.ds(grp_base(0), IDXSPAN)],
                        gidx_v.at[pl.ds(0, IDXSPAN)])
        pltpu.async_copy(rows_hbm.at[pl.ds(grp_base(1), IDXSPAN)],
                        gidx_v.at[pl.ds(IDXSPAN, IDXSPAN)], sem_gi)
        issue_gather(0)
        issue_gather(1)

        def inner(g, j):
            nidx = g * GB + j
            boff = (nidx % 3) * 64
            d, a, skew, m, a_rel, use_grp = node_params(nidx)
            rank = jnp.maximum((d - 1) >> 1, 0)

            @pl.when(use_grp)
            def _():
                pltpu.make_async_copy(h_hbm.at[pl.ds(0, 64)],
                                      vals_v.at[pl.ds(0, 64)],
                                      sem_gv).wait()

            @pl.when((j == GB - 2) & (g + 1 < NG))
            def _():
                pltpu.make_async_copy(rows_hbm.at[pl.ds(0, IDXSPAN)],
                                      gidx_v.at[pl.ds(0, IDXSPAN)],
                                      sem_gi).wait()

            @pl.when((j == GB - 1) & (g + 2 < NG))
            def _():
                pltpu.async_copy(
                    rows_hbm.at[pl.ds(grp_base(g + 2), IDXSPAN)],
                    gidx_v.at[pl.ds(pl.multiple_of(
                        (g & 1) * IDXSPAN, 8), IDXSPAN)], sem_gi)

            @pl.when(nidx + 2 < NPW)
            def _():
                issue_gather(nidx + 2)

            def sort_branch(B):
                net = _NETS[B]

                def run():
                    def fg_body(fg, carry):
                        lo = fg * LANES
                        w = []
                        for r in range(B):
                            v = vals_v[boff + r, pl.ds(lo, LANES)]
                            keep = (r >= skew) & (r < m)
                            w.append(jnp.where(keep, v, inf16))
                        for (i, jj) in net:
                            lo_v = jnp.minimum(w[i], w[jj])
                            hi_v = jnp.maximum(w[i], w[jj])
                            w[i] = lo_v
                            w[jj] = hi_v
                        for r in range(B // 2):
                            srt_v[r, :] = w[r]
                        med = srt_v[rank, :]
                        outg_v[j, pl.ds(lo, LANES)] = (
                            med + bias_v[pl.ds(lo, LANES)])
                        return carry

                    lax.fori_loop(0, NFG, fg_body, 0)
                    return jnp.int32(0)

                return run

            def fallback():
                nch = (m + 63) >> 6
                zero8 = tuple(zero_i for _ in range(NFG))
                p0 = tuple(zero_u for _ in range(NFG))

                def round_body(tt, ps):
                    t = (31 - tt).astype(jnp.uint32)
                    cands = tuple(p | (jnp.uint32(1) << t) for p in ps)

                    def chunk_body(c, cnts):
                        pltpu.sync_copy(
                            rows_hbm.at[
                                pl.ds(pl.multiple_of(a + c * 64, 8), 64)],
                            fidx_v)
                        pltpu.async_copy(h_hbm.at[fidx_v],
                                         fvals_v, sem_f).wait()
                        base_r = c * 64

                        def row_blk(rb, cnts):
                            cl = list(cnts)
                            for rr in range(4):
                                r = rb * 4 + rr
                                gr = base_r + r
                                keep = (gr >= skew) & (gr < m)
                                for fg in range(NFG):
                                    v = fvals_v[r, pl.ds(fg * LANES, LANES)]
                                    b = lax.bitcast_convert_type(
                                        v, jnp.int32)
                                    sgn = b >> 31
                                    key = (lax.bitcast_convert_type(
                                               b, jnp.uint32)
                                           ^ (lax.bitcast_convert_type(
                                                  sgn, jnp.uint32)
                                              | jnp.uint32(0x80000000)))
                                    key = jnp.where(keep, key, ffff_u)
                                    hit = jnp.where(
                                        key < cands[fg], one_i, zero_i)
                                    cl[fg] = cl[fg] + hit
                            return tuple(cl)

                        return lax.fori_loop(0, 16, row_blk, cnts)

                    cnts = lax.fori_loop(0, nch, chunk_body, zero8)
                    return tuple(
                        jnp.where(cnt <= rank, cand, p)
                        for (cnt, cand, p) in zip(cnts, cands, ps))

                ps = lax.fori_loop(0, 32, round_body, p0)
                for fg in range(NFG):
                    key = ps[fg]
                    pos = (key >> jnp.uint32(31)) > jnp.uint32(0)
                    bu = jnp.where(pos, key ^ jnp.uint32(0x80000000), ~key)
                    med = lax.bitcast_convert_type(bu, jnp.float32)
                    outg_v[j, pl.ds(fg * LANES, LANES)] = (
                        med + bias_v[pl.ds(fg * LANES, LANES)])
                return jnp.int32(0)

            bkt = jnp.where(m <= 32, 0, jnp.where(m <= 48, 1, 2))

            @pl.when(use_grp)
            def _():
                lax.switch(bkt, [sort_branch(32), sort_branch(48),
                                 sort_branch(64)])

            @pl.when(jnp.logical_not(use_grp))
            def _():
                fallback()

            @pl.when(j == GB - 1)
            def _():
                pltpu.sync_copy(outg_v,
                                out_hbm.at[pl.ds(nbase + g * GB, GB)])

        def gloop(g, carry):
            def jloop(j, c2):
                inner(g, j)
                return c2

            lax.fori_loop(0, GB, jloop, 0)
            return carry

        lax.fori_loop(0, NG, gloop, 0)

    return kern(h, rows_pad, offs_pad, deg_pad, bias)


def kernel(x, edge_index, W, bias):
    N = x.shape[0]
    E = edge_index.shape[1]
    E2 = E + N
    loops = jnp.arange(N, dtype=edge_index.dtype)
    row = jnp.concatenate([edge_index[0], loops])
    col = jnp.concatenate([edge_index[1], loops])
    keys = jnp.sort((col.astype(jnp.int32) << 14) | row.astype(jnp.int32))
    row_s = keys & jnp.int32(16383)
    bounds = jnp.searchsorted(
        keys, jnp.arange(N + 1, dtype=jnp.int32) << 14, side="left"
    ).astype(jnp.int32)
    offs = bounds[:N]
    deg = bounds[1:] - bounds[:-1]

    npad = NW * NPW
    offs_pad = jnp.concatenate(
        [offs, jnp.full((npad - N,), E2, jnp.int32)])
    deg_pad = jnp.concatenate(
        [deg, jnp.zeros((npad - N,), jnp.int32)])
    rows_pad = jnp.concatenate(
        [row_s, jnp.zeros((IDXSPAN + 128,), jnp.int32)])

    xp = jnp.pad(x, ((0, npad - N), (0, 0)))
    h = _matmul(xp, W)

    out = _sc_median(h, rows_pad, offs_pad, deg_pad, bias)
    return out[:N]


# speedup vs baseline: 1.7286x; 1.7286x over previous
import functools

import jax
import jax.numpy as jnp
from jax import lax
from jax.experimental import pallas as pl
from jax.experimental.pallas import tpu as pltpu
from jax.experimental.pallas import tpu_sc as plsc

NW = 32
NPW = 320
GB = 16
IDXSPAN = 2048
LANES = 16
D = 128
NFG = D // LANES


def _oddeven_network(length):
    pairs = []

    def merge(lo, n, r):
        step = r * 2
        if step < n:
            merge(lo, n, step)
            merge(lo + r, n, step)
            for i in range(lo + r, lo + n - r, step):
                pairs.append((i, i + r))
        else:
            pairs.append((lo, lo + r))

    def sort(lo, n):
        if n > 1:
            m = n // 2
            sort(lo, m)
            sort(lo + m, m)
            merge(lo, n, 1)

    sort(0, length)
    return pairs


def _prune_for_median(net, B):
    needed = set(range((B + 1) // 2))
    keep = []
    for (i, j) in reversed(net):
        if i in needed or j in needed:
            keep.append((i, j))
            needed.add(i)
            needed.add(j)
    keep.reverse()
    return keep


_NET64 = _oddeven_network(64)
_NET48 = [(i, j) for (i, j) in _NET64 if j < 48]
_NETS = {32: _prune_for_median(_oddeven_network(32), 32),
         48: _prune_for_median(_NET48, 48),
         64: _prune_for_median(_NET64, 64)}


def _matmul(x, W):
    rows = x.shape[0]

    def body(x_ref, w_ref, o_ref):
        o_ref[...] = jnp.dot(x_ref[...], w_ref[...],
                             preferred_element_type=jnp.float32)

    return pl.pallas_call(
        body,
        grid=(rows // 128,),
        in_specs=[pl.BlockSpec((128, x.shape[1]), lambda i: (i, 0)),
                  pl.BlockSpec(W.shape, lambda i: (0, 0))],
        out_specs=pl.BlockSpec((128, W.shape[1]), lambda i: (i, 0)),
        out_shape=jax.ShapeDtypeStruct((rows, W.shape[1]), jnp.float32),
    )(x, W)


def _sc_median(h, rows_pad, offs_pad, deg_pad, bias):
    npad = offs_pad.shape[0]
    NG = NPW // GB
    mesh = plsc.VectorSubcoreMesh(core_axis_name="c", subcore_axis_name="s",
                                  num_cores=2, num_subcores=16)

    @functools.partial(
        pl.kernel, mesh=mesh,
        out_type=jax.ShapeDtypeStruct((npad, D), jnp.float32),
        scratch_types=[
            pltpu.VMEM((NPW + 16,), jnp.int32),
            pltpu.VMEM((NPW + 16,), jnp.int32),
            pltpu.VMEM((2 * IDXSPAN,), jnp.int32),
            pltpu.VMEM((192, D), jnp.float32),
            pltpu.VMEM((64,), jnp.int32),
            pltpu.VMEM((64, D), jnp.float32),
            pltpu.VMEM((32, LANES), jnp.float32),
            pltpu.VMEM((GB, D), jnp.float32),
            pltpu.VMEM((D,), jnp.float32),
            pltpu.SemaphoreType.DMA,
            pltpu.SemaphoreType.DMA,
            pltpu.SemaphoreType.DMA,
        ])
    def kern(h_hbm, rows_hbm, offs_hbm, deg_hbm, bias_hbm, out_hbm,
             offs_v, deg_v, gidx_v, vals_v, fidx_v, fvals_v, srt_v,
             outg_v, bias_v, sem_gi, sem_gv, sem_f):
        cid = lax.axis_index("c")
        sid = lax.axis_index("s")
        wid = sid * 2 + (1 - cid)
        nbase = pl.multiple_of(wid * NPW, 8)
        pltpu.sync_copy(offs_hbm.at[pl.ds(nbase, NPW)],
                        offs_v.at[pl.ds(0, NPW)])
        pltpu.sync_copy(deg_hbm.at[pl.ds(nbase, NPW)],
                        deg_v.at[pl.ds(0, NPW)])
        pltpu.sync_copy(bias_hbm, bias_v)
        zf = bias_v[pl.ds(0, LANES)] * 0.0
        inf16 = zf + jnp.float32(jnp.inf)
        zero_i = lax.bitcast_convert_type(zf, jnp.int32)
        one_i = zero_i + 1
        zero_u = lax.bitcast_convert_type(zf, jnp.uint32)
        ffff_u = ~zero_u

        def rd(ref, i):
            return ref[pl.ds(i, 16)][0]

        def grp_base(g):
            return pl.multiple_of((rd(offs_v, g * GB) >> 3) << 3, 8)

        def node_params(nidx):
            start = rd(offs_v, nidx)
            d = rd(deg_v, nidx)
            a = pl.multiple_of((start >> 3) << 3, 8)
            skew = start - a
            m = skew + d
            a_rel = pl.multiple_of(a - grp_base(nidx >> 4), 8)
            use_grp = (m <= 64) & ((a_rel + 64) <= IDXSPAN)
            return d, a, skew, m, a_rel, use_grp

        def issue_gather(nidx):
            d, a, skew, m, a_rel, use_grp = node_params(nidx)

            @pl.when(use_grp)
            def _():
                goff = ((nidx >> 4) & 1) * IDXSPAN
                boff = (nidx % 3) * 64
                pltpu.async_copy(
                    h_hbm.at[gidx_v.at[
                        pl.ds(pl.multiple_of(goff + a_rel, 8), 64)]],
                    vals_v.at[pl.ds(pl.multiple_of(boff, 8), 64)],
                    sem_gv)

        pltpu.sync_copy(rows_hbm.at[pl.ds(grp_base(0), IDXSPAN)],
                        gidx_v.at[pl.ds(0, IDXSPAN)])
        pltpu.async_copy(rows_hbm.at[pl.ds(grp_base(1), IDXSPAN)],
                        gidx_v.at[pl.ds(IDXSPAN, IDXSPAN)], sem_gi)
        issue_gather(0)
        issue_gather(1)

        def inner(g, j):
            nidx = g * GB + j
            boff = (nidx % 3) * 64
            d, a, skew, m, a_rel, use_grp = node_params(nidx)
            rank = jnp.maximum((d - 1) >> 1, 0)

            @pl.when(use_grp)
            def _():
                pltpu.make_async_copy(h_hbm.at[pl.ds(0, 64)],
                                      vals_v.at[pl.ds(0, 64)],
                                      sem_gv).wait()

            @pl.when((j == GB - 2) & (g + 1 < NG))
            def _():
                pltpu.make_async_copy(rows_hbm.at[pl.ds(0, IDXSPAN)],
                                      gidx_v.at[pl.ds(0, IDXSPAN)],
                                      sem_gi).wait()

            @pl.when((j == GB - 1) & (g + 2 < NG))
            def _():
                pltpu.async_copy(
                    rows_hbm.at[pl.ds(grp_base(g + 2), IDXSPAN)],
                    gidx_v.at[pl.ds(pl.multiple_of(
                        (g & 1) * IDXSPAN, 8), IDXSPAN)], sem_gi)

            @pl.when(nidx + 2 < NPW)
            def _():
                issue_gather(nidx + 2)

            def sort_branch(B):
                net = _NETS[B]

                def run():
                    def fg_body(fg, carry):
                        lo = fg * LANES
                        w = []
                        for r in range(B):
                            v = vals_v[boff + r, pl.ds(lo, LANES)]
                            keep = (r >= skew) & (r < m)
                            w.append(jnp.where(keep, v, inf16))
                        for (i, jj) in net:
                            lo_v = jnp.minimum(w[i], w[jj])
                            hi_v = jnp.maximum(w[i], w[jj])
                            w[i] = lo_v
                            w[jj] = hi_v
                        for r in range(B // 2):
                            srt_v[r, :] = w[r]
                        med = srt_v[rank, :]
                        outg_v[j, pl.ds(lo, LANES)] = (
                            med + bias_v[pl.ds(lo, LANES)])
                        return carry

                    lax.fori_loop(0, NFG, fg_body, 0)
                    return jnp.int32(0)

                return run

            def fallback():
                nch = (m + 63) >> 6
                zero8 = tuple(zero_i for _ in range(NFG))
                p0 = tuple(zero_u for _ in range(NFG))

                def round_body(tt, ps):
                    t = (31 - tt).astype(jnp.uint32)
                    cands = tuple(p | (jnp.uint32(1) << t) for p in ps)

                    def chunk_body(c, cnts):
                        pltpu.sync_copy(
                            rows_hbm.at[
                                pl.ds(pl.multiple_of(a + c * 64, 8), 64)],
                            fidx_v)
                        pltpu.async_copy(h_hbm.at[fidx_v],
                                         fvals_v, sem_f).wait()
                        base_r = c * 64

                        def row_blk(rb, cnts):
                            cl = list(cnts)
                            for rr in range(4):
                                r = rb * 4 + rr
                                gr = base_r + r
                                keep = (gr >= skew) & (gr < m)
                                for fg in range(NFG):
                                    v = fvals_v[r, pl.ds(fg * LANES, LANES)]
                                    b = lax.bitcast_convert_type(
                                        v, jnp.int32)
                                    sgn = b >> 31
                                    key = (lax.bitcast_convert_type(
                                               b, jnp.uint32)
                                           ^ (lax.bitcast_convert_type(
                                                  sgn, jnp.uint32)
                                              | jnp.uint32(0x80000000)))
                                    key = jnp.where(keep, key, ffff_u)
                                    hit = jnp.where(
                                        key < cands[fg], one_i, zero_i)
                                    cl[fg] = cl[fg] + hit
                            return tuple(cl)

                        return lax.fori_loop(0, 16, row_blk, cnts)

                    cnts = lax.fori_loop(0, nch, chunk_body, zero8)
                    return tuple(
                        jnp.where(cnt <= rank, cand, p)
                        for (cnt, cand, p) in zip(cnts, cands, ps))

                ps = lax.fori_loop(0, 32, round_body, p0)
                for fg in range(NFG):
                    key = ps[fg]
                    pos = (key >> jnp.uint32(31)) > jnp.uint32(0)
                    bu = jnp.where(pos, key ^ jnp.uint32(0x80000000), ~key)
                    med = lax.bitcast_convert_type(bu, jnp.float32)
                    outg_v[j, pl.ds(fg * LANES, LANES)] = (
                        med + bias_v[pl.ds(fg * LANES, LANES)])
                return jnp.int32(0)

            bkt = jnp.where(m <= 32, 0, jnp.where(m <= 48, 1, 2))

            @pl.when(use_grp)
            def _():
                lax.switch(bkt, [sort_branch(32), sort_branch(48),
                                 sort_branch(64)])

            @pl.when(jnp.logical_not(use_grp))
            def _():
                fallback()

            @pl.when(j == GB - 1)
            def _():
                pltpu.sync_copy(outg_v,
                                out_hbm.at[pl.ds(nbase + g * GB, GB)])

        def gloop(g, carry):
            def jloop(j, c2):
                inner(g, j)
                return c2

            lax.fori_loop(0, GB, jloop, 0)
            return carry

        lax.fori_loop(0, NG, gloop, 0)

    return kern(h, rows_pad, offs_pad, deg_pad, bias)


def kernel(x, edge_index, W, bias):
    N = x.shape[0]
    E = edge_index.shape[1]
    E2 = E + N
    loops = jnp.arange(N, dtype=edge_index.dtype)
    row = jnp.concatenate([edge_index[0], loops])
    col = jnp.concatenate([edge_index[1], loops])
    _, row_s = lax.sort((col, row), num_keys=1, is_stable=False)
    row_s = row_s.astype(jnp.int32)
    deg = jnp.bincount(col, length=N).astype(jnp.int32)
    offs = (jnp.cumsum(deg) - deg).astype(jnp.int32)

    npad = NW * NPW
    offs_pad = jnp.concatenate(
        [offs, jnp.full((npad - N,), E2, jnp.int32)])
    deg_pad = jnp.concatenate(
        [deg, jnp.zeros((npad - N,), jnp.int32)])
    rows_pad = jnp.concatenate(
        [row_s, jnp.zeros((IDXSPAN + 128,), jnp.int32)])

    xp = jnp.pad(x, ((0, npad - N), (0, 0)))
    h = _matmul(xp, W)

    out = _sc_median(h, rows_pad, offs_pad, deg_pad, bias)
    return out[:N]
